# Initial kernel scaffold; baseline (speedup 1.0000x reference)
import jax
import jax.numpy as jnp
from jax.experimental import pallas as pl


def kernel(input, target, emb_table, W, b):
    raise NotImplementedError("write your pallas kernel here")


# speedup vs baseline: 1.0155x; 1.0155x over previous
import functools

import jax
import jax.numpy as jnp
from jax import lax
from jax.experimental import pallas as pl
from jax.experimental.pallas import tpu as pltpu
from jax.experimental.pallas import tpu_sc as plsc

NC = 2
NS = 16
L = 16
NW = NC * NS


def _mtab_body(emb_ref, wt_ref, b_ref, m_ref, lse_ref):
    m = jax.lax.dot_general(
        emb_ref[...], wt_ref[...],
        (((1,), (0,)), ((), ())),
        preferred_element_type=jnp.float32,
    ) + b_ref[...]
    m_ref[...] = m
    mx = jnp.max(m, axis=1, keepdims=True)
    lse_ref[...] = mx + jnp.log(jnp.sum(jnp.exp(m - mx), axis=1, keepdims=True))


def _make_mtab(V, E):
    return pl.pallas_call(
        _mtab_body,
        out_shape=(
            jax.ShapeDtypeStruct((V, V), jnp.float32),
            jax.ShapeDtypeStruct((V, 1), jnp.float32),
        ),
    )


def _make_gather(V, NTOK):
    TPT = NTOK // NW
    CH = 64
    assert TPT % CH == 0 and CH % L == 0
    mesh = plsc.VectorSubcoreMesh(core_axis_name="c", subcore_axis_name="s")

    @functools.partial(
        pl.kernel,
        mesh=mesh,
        compiler_params=pltpu.CompilerParams(
            use_tc_tiling_on_sc=False, needs_layout_passes=False),
        out_type=(
            jax.ShapeDtypeStruct((NTOK, V), jnp.float32),
            jax.ShapeDtypeStruct((NW * L,), jnp.float32),
        ),
        scratch_types=[
            pltpu.VMEM((TPT,), jnp.int32),
            pltpu.VMEM((TPT,), jnp.int32),
            pltpu.VMEM((V,), jnp.float32),
            pltpu.VMEM((CH, V), jnp.float32),
            pltpu.VMEM((L,), jnp.float32),
            pltpu.SemaphoreType.DMA,
        ],
    )
    def gather_k(m_hbm, idx_hbm, tgt_hbm, lse_hbm, out_hbm, part_hbm,
                 idx_v, tgt_v, lse_v, rows_v, acc_v, sem):
        wid = lax.axis_index("s") * NC + lax.axis_index("c")
        base = wid * TPT
        pltpu.sync_copy(idx_hbm.at[pl.ds(base, TPT)], idx_v)
        pltpu.sync_copy(tgt_hbm.at[pl.ds(base, TPT)], tgt_v)
        pltpu.sync_copy(lse_hbm, lse_v)
        acc = jnp.zeros((L,), jnp.float32)
        for c in range(TPT // CH):
            pltpu.async_copy(
                m_hbm.at[idx_v.at[pl.ds(c * CH, CH)]], rows_v, sem).wait()
            pltpu.sync_copy(rows_v, out_hbm.at[pl.ds(base + c * CH, CH)])
            for g in range(CH // L):
                off = c * CH + g * L
                toks = idx_v[pl.ds(off, L)]
                tgts = tgt_v[pl.ds(off, L)]
                rid = lax.iota(jnp.int32, L) + (g * L)
                tlogit = plsc.load_gather(rows_v, [rid, tgts])
                ltok = plsc.load_gather(lse_v, [toks])
                acc = acc + (ltok - tlogit)
        acc_v[...] = acc
        pltpu.sync_copy(acc_v, part_hbm.at[pl.ds(wid * L, L)])

    return gather_k


def _loss_body(p_ref, o_ref, *, ntok):
    o_ref[...] = jnp.sum(p_ref[...], keepdims=True) * (1.0 / ntok)


def _make_loss(ntok):
    return pl.pallas_call(
        functools.partial(_loss_body, ntok=ntok),
        out_shape=jax.ShapeDtypeStruct((1, 1), jnp.float32),
    )


def kernel(input, target, emb_table, W, b):
    Bv, Sv = input.shape
    V, E = emb_table.shape
    NTOK = Bv * Sv

    m, lse = _make_mtab(V, E)(emb_table, W.T, b.reshape(1, V))
    idx = input.reshape(NTOK)
    tgt = target.reshape(NTOK)
    logits_flat, part = _make_gather(V, NTOK)(m, idx, tgt, lse.reshape(V))
    loss2d = _make_loss(NTOK)(part.reshape(NW, L))
    return logits_flat.reshape(Bv, Sv, V), loss2d[0, 0]


# speedup vs baseline: 1.0331x; 1.0174x over previous
import functools

import jax
import jax.numpy as jnp
from jax import lax
from jax.experimental import pallas as pl
from jax.experimental.pallas import tpu as pltpu
from jax.experimental.pallas import tpu_sc as plsc

NC = 2
NS = 16
L = 16
NW = NC * NS


def _mtab_body(emb_ref, wt_ref, b_ref, m_ref, lse_ref):
    m = jax.lax.dot_general(
        emb_ref[...], wt_ref[...],
        (((1,), (1,)), ((), ())),
        preferred_element_type=jnp.float32,
    ) + b_ref[...]
    m_ref[...] = m
    mx = jnp.max(m, axis=1, keepdims=True)
    lse_ref[...] = mx + jnp.log(jnp.sum(jnp.exp(m - mx), axis=1, keepdims=True))


def _make_mtab(V, E):
    return pl.pallas_call(
        _mtab_body,
        out_shape=(
            jax.ShapeDtypeStruct((V, V), jnp.float32),
            jax.ShapeDtypeStruct((V, 1), jnp.float32),
        ),
    )


def _make_gather(V, Bv, Sv):
    NTOK = Bv * Sv
    TPT = NTOK // NW
    TPS = Sv // TPT
    CH = 64
    assert TPT % CH == 0 and CH % L == 0 and Sv % TPT == 0
    mesh = plsc.VectorSubcoreMesh(core_axis_name="c", subcore_axis_name="s")

    @functools.partial(
        pl.kernel,
        mesh=mesh,
        compiler_params=pltpu.CompilerParams(
            use_tc_tiling_on_sc=False, needs_layout_passes=False),
        out_type=(
            jax.ShapeDtypeStruct((Bv, Sv, V), jnp.float32),
            jax.ShapeDtypeStruct((NW * L,), jnp.float32),
        ),
        scratch_types=[
            pltpu.VMEM((TPT,), jnp.int32),
            pltpu.VMEM((TPT,), jnp.int32),
            pltpu.VMEM((V,), jnp.float32),
            pltpu.VMEM((CH, V), jnp.float32),
            pltpu.VMEM((L,), jnp.float32),
            pltpu.SemaphoreType.DMA,
        ],
    )
    def gather_k(m_hbm, idx_hbm, tgt_hbm, lse_hbm, out_hbm, part_hbm,
                 idx_v, tgt_v, lse_v, rows_v, acc_v, sem):
        wid = lax.axis_index("s") * NC + lax.axis_index("c")
        base = wid * TPT
        bpos = wid // TPS
        spos = (wid % TPS) * TPT
        pltpu.sync_copy(idx_hbm.at[pl.ds(base, TPT)], idx_v)
        pltpu.sync_copy(tgt_hbm.at[pl.ds(base, TPT)], tgt_v)
        pltpu.sync_copy(lse_hbm, lse_v)
        acc = jnp.zeros((L,), jnp.float32)
        for c in range(TPT // CH):
            pltpu.async_copy(
                m_hbm.at[idx_v.at[pl.ds(c * CH, CH)]], rows_v, sem).wait()
            pltpu.sync_copy(rows_v, out_hbm.at[bpos, pl.ds(spos + c * CH, CH)])
            for g in range(CH // L):
                off = c * CH + g * L
                toks = idx_v[pl.ds(off, L)]
                tgts = tgt_v[pl.ds(off, L)]
                rid = lax.iota(jnp.int32, L) + (g * L)
                tlogit = plsc.load_gather(rows_v, [rid, tgts])
                ltok = plsc.load_gather(lse_v, [toks])
                acc = acc + (ltok - tlogit)
        acc_v[...] = acc
        pltpu.sync_copy(acc_v, part_hbm.at[pl.ds(wid * L, L)])

    return gather_k


def _loss_body(p_ref, o_ref, *, ntok):
    o_ref[...] = jnp.sum(p_ref[...], keepdims=True) * (1.0 / ntok)


def _make_loss(ntok):
    return pl.pallas_call(
        functools.partial(_loss_body, ntok=ntok),
        out_shape=jax.ShapeDtypeStruct((1, 1), jnp.float32),
    )


def kernel(input, target, emb_table, W, b):
    Bv, Sv = input.shape
    V, E = emb_table.shape
    NTOK = Bv * Sv

    m, lse = _make_mtab(V, E)(emb_table, W, b.reshape(1, V))
    idx = input.reshape(NTOK)
    tgt = target.reshape(NTOK)
    logits, part = _make_gather(V, Bv, Sv)(m, idx, tgt, lse.reshape(V))
    loss2d = _make_loss(NTOK)(part.reshape(NW, L))
    return logits, loss2d[0, 0]
